# baseline (speedup 1.0000x reference)
import functools

import jax
import jax.numpy as jnp
from jax import lax
from jax.experimental import pallas as pl
from jax.experimental.pallas import tpu as pltpu
from jax.experimental.pallas import tpu_sc as plsc

CHUNK = 128


@functools.cache
def _make_gather(batch, vocab, dim):
    info = plsc.get_sparse_core_info()
    nc, ns = info.num_cores, info.num_subcores
    nw = nc * ns
    assert batch % (nw * CHUNK) == 0
    b_per_w = batch // nw
    n_chunks = b_per_w // CHUNK

    mesh = plsc.VectorSubcoreMesh(core_axis_name="c", subcore_axis_name="s")

    @functools.partial(
        pl.kernel,
        mesh=mesh,
        out_type=jax.ShapeDtypeStruct((batch, dim), jnp.float32),
        scratch_types=[
            pltpu.VMEM((n_chunks, CHUNK), jnp.int32),
            pltpu.VMEM((b_per_w, dim), jnp.float32),
            pltpu.SemaphoreType.DMA,
        ],
        compiler_params=pltpu.CompilerParams(use_tc_tiling_on_sc=False),
    )
    def k(idx_hbm, table_hbm, out_hbm, idx_v, rows_v, sem):
        wid = lax.axis_index("s") * nc + lax.axis_index("c")
        pltpu.sync_copy(idx_hbm.at[wid], idx_v)
        copies = [
            pltpu.async_copy(
                table_hbm.at[idx_v.at[j]],
                rows_v.at[pl.ds(j * CHUNK, CHUNK)],
                sem,
            )
            for j in range(n_chunks)
        ]
        for c in copies:
            c.wait()
        pltpu.sync_copy(rows_v, out_hbm.at[pl.ds(wid * b_per_w, b_per_w)])

    return k


def kernel(unique_ids, table):
    batch, = unique_ids.shape
    vocab, dim = table.shape
    info = plsc.get_sparse_core_info()
    nw = info.num_cores * info.num_subcores
    idx = unique_ids.astype(jnp.int32).reshape(nw, batch // (nw * CHUNK), CHUNK)
    return _make_gather(batch, vocab, dim)(idx, table)


# speedup vs baseline: 2.4357x; 2.4357x over previous
import functools

import jax
import jax.numpy as jnp
from jax import lax
from jax.experimental import pallas as pl
from jax.experimental.pallas import tpu as pltpu
from jax.experimental.pallas import tpu_sc as plsc

PANEL_W = 512
RING = 128
L = 16


@functools.cache
def _make_gather(batch, vocab, dim):
    info = plsc.get_sparse_core_info()
    nc, ns = info.num_cores, info.num_subcores
    nw = nc * ns
    n_full = vocab // PANEL_W
    tail_w = vocab - n_full * PANEL_W
    per, rem = divmod(n_full, nw)
    out_rows = batch + nw
    assert out_rows % 8 == 0 and batch % L == 0

    mesh = plsc.VectorSubcoreMesh(core_axis_name="c", subcore_axis_name="s")

    @functools.partial(
        pl.kernel,
        mesh=mesh,
        out_type=jax.ShapeDtypeStruct((out_rows, 2 * dim), jnp.float32),
        scratch_types=[
            pltpu.VMEM((batch,), jnp.int32),
            pltpu.VMEM((batch + L,), jnp.int32),
            pltpu.VMEM((batch + L,), jnp.int32),
            pltpu.VMEM((dim, PANEL_W), jnp.float32),
            pltpu.VMEM((dim, max(tail_w, 1)), jnp.float32),
            pltpu.VMEM((RING, 2 * dim), jnp.float32),
            pltpu.VMEM((1, RING), jnp.int32),
            pltpu.VMEM((L,), jnp.int32),
            pltpu.VMEM((L,), jnp.int32),
            pltpu.SemaphoreType.DMA,
        ],
        compiler_params=pltpu.CompilerParams(use_tc_tiling_on_sc=True, needs_layout_passes=False),
    )
    def k(idx_hbm, tt_hbm, tail_hbm, out_hbm,
          idx_v, bkt_i, bkt_b, panel_v, tail_v, ring_v, rd_v, st_c, st_b,
          sem):
        wid = lax.axis_index("s") * nc + lax.axis_index("c")
        iota = lax.broadcasted_iota(jnp.int32, (L,), 0)
        zeros = jnp.zeros((L,), jnp.int32)
        dummy = jnp.full((L,), batch + wid, jnp.int32)
        lane0 = iota == 0

        n_my = per + jnp.where(wid < rem, 1, 0)
        p0 = wid * per + jnp.minimum(wid, rem)
        lo = p0 * PANEL_W
        hi = lo + n_my * PANEL_W
        hi = jnp.where(wid == nw - 1, vocab, hi)

        pltpu.sync_copy(idx_hbm, idx_v)

        def reset_rd():
            for g in range(RING // L):
                plsc.store_scatter(rd_v.at[...], [zeros, iota + g * L], dummy)

        reset_rd()

        def scan_body(kk, blen):
            iv = idx_v[pl.ds(kk * L, L)]
            bv = iota + kk * L
            m = (iv >= lo) & (iv < hi)
            plsc.store_compressed(bkt_i.at[pl.ds(blen, L)], iv, mask=m)
            plsc.store_compressed(bkt_b.at[pl.ds(blen, L)], bv, mask=m)
            return blen + plsc.all_reduce_population_count(m)[0]

        blen = lax.fori_loop(0, batch // L, scan_body, jnp.int32(0))
        nk = (blen + L - 1) // L

        def flush(rp):
            pltpu.sync_copy(ring_v, out_hbm.at[rd_v.at[0]])
            reset_rd()
            return rp

        def extract(panel_ref, off, width, rp):
            def rescan_body(kk, rp):
                iv = bkt_i[pl.ds(kk * L, L)]
                bv = bkt_b[pl.ds(kk * L, L)]
                valid = (iota + kk * L) < blen
                m = valid & (iv >= off) & (iv < off + width)
                cnt = plsc.all_reduce_population_count(m)[0]
                plsc.store_compressed(st_c.at[...], iv - off, mask=m)
                plsc.store_compressed(st_b.at[...], bv, mask=m)

                def match_body(t, rp):
                    tv = jnp.full((L,), t, jnp.int32)
                    cvec = plsc.load_gather(st_c.at[...], [tv])
                    bvec = plsc.load_gather(st_b.at[...], [tv])
                    rpv = jnp.full((L,), rp, jnp.int32)
                    for g in range(dim // L):
                        dvec = iota + g * L
                        vals = plsc.load_gather(panel_ref.at[...], [dvec, cvec])
                        plsc.store_scatter(ring_v.at[...], [rpv, dvec], vals)
                    plsc.store_scatter(rd_v.at[...], [zeros, rpv], bvec,
                                       mask=lane0)
                    rp = rp + 1

                    @pl.when(rp == RING)
                    def _():
                        flush(rp)

                    return jnp.where(rp == RING, 0, rp)

                return lax.fori_loop(0, cnt, match_body, rp)

            return lax.fori_loop(0, nk, rescan_body, rp)

        def panel_body(p, rp):
            off = pl.multiple_of((p0 + p) * PANEL_W, PANEL_W)
            pltpu.sync_copy(tt_hbm.at[:, pl.ds(off, PANEL_W)], panel_v)
            return extract(panel_v, off, PANEL_W, rp)

        rp = lax.fori_loop(0, n_my, panel_body, jnp.int32(0))

        if tail_w:
            @pl.when(wid == nw - 1)
            def _():
                pltpu.sync_copy(tail_hbm, tail_v)

            eff_w = jnp.where(wid == nw - 1, tail_w, 0)
            rp = extract(tail_v, jnp.int32(n_full * PANEL_W), eff_w, rp)

        flush(rp)

    return k


def kernel(unique_ids, table):
    batch, = unique_ids.shape
    vocab, dim = table.shape
    tail_start = (vocab // PANEL_W) * PANEL_W
    idx = unique_ids.astype(jnp.int32)
    tt = table.T
    tail = table[tail_start:].T if tail_start < vocab else table[:1].T
    out_wide = _make_gather(batch, vocab, dim)(idx, tt, tail)
    return out_wide[:batch, :dim]
